# Initial kernel scaffold; baseline (speedup 1.0000x reference)
import jax
import jax.numpy as jnp
from jax.experimental import pallas as pl


def kernel(indices, table):
    raise NotImplementedError("write your pallas kernel here")


# speedup vs baseline: 1.0933x; 1.0933x over previous
import functools

import jax
import jax.numpy as jnp
from jax import lax
from jax.experimental import pallas as pl
from jax.experimental.pallas import tpu as pltpu
from jax.experimental.pallas import tpu_sc as plsc


@functools.lru_cache(maxsize=None)
def _make_gather(B, D, C):
    info = plsc.get_sparse_core_info()
    NW = info.num_cores * info.num_subcores
    assert B % (NW * C) == 0 and C % 8 == 0
    b_per_w = B // NW
    n_chunks = b_per_w // C
    mesh = plsc.VectorSubcoreMesh(core_axis_name="c", subcore_axis_name="s")

    @functools.partial(
        pl.kernel,
        mesh=mesh,
        out_type=jax.ShapeDtypeStruct((B, D), jnp.float32),
        scratch_types=[
            pltpu.VMEM((C,), jnp.int32),
            pltpu.VMEM((C, D), jnp.float32),
            pltpu.SemaphoreType.DMA,
        ],
        compiler_params=pltpu.CompilerParams(use_tc_tiling_on_sc=False),
    )
    def gather_kernel(idx_hbm, table_hbm, out_hbm, idx_v, rows_v, gsem):
        wid = lax.axis_index("s") * info.num_cores + lax.axis_index("c")
        base = wid * b_per_w

        @pl.loop(0, n_chunks)
        def _chunk(g):
            off = pl.multiple_of(base + g * C, 8)
            pltpu.sync_copy(idx_hbm.at[pl.ds(off, C)], idx_v)
            pltpu.async_copy(table_hbm.at[idx_v], rows_v, gsem).wait()
            pltpu.sync_copy(rows_v, out_hbm.at[pl.ds(off, C)])

    return gather_kernel


def kernel(indices, table):
    B, H = indices.shape
    V, D = table.shape
    flat_idx = indices.reshape(B * H).astype(jnp.int32)
    out = _make_gather(B * H, D, 1024)(flat_idx, table)
    return out.reshape(B, H, D)


# speedup vs baseline: 1.1121x; 1.0172x over previous
import functools

import jax
import jax.numpy as jnp
from jax import lax
from jax.experimental import pallas as pl
from jax.experimental.pallas import tpu as pltpu
from jax.experimental.pallas import tpu_sc as plsc


@functools.lru_cache(maxsize=None)
def _make_gather(B, D, C):
    info = plsc.get_sparse_core_info()
    NC = info.num_cores
    NW = NC * info.num_subcores
    assert B % (NW * C) == 0 and C % 8 == 0
    b_per_w = B // NW
    n_chunks = b_per_w // C
    assert n_chunks >= 2 and n_chunks % 2 == 0
    mesh = plsc.VectorSubcoreMesh(core_axis_name="c", subcore_axis_name="s")

    @functools.partial(
        pl.kernel,
        mesh=mesh,
        out_type=jax.ShapeDtypeStruct((B, D), jnp.float32),
        scratch_types=[
            pltpu.VMEM((n_chunks, C), jnp.int32),
            pltpu.VMEM((2, C, D), jnp.float32),
            pltpu.SemaphoreType.DMA,
            pltpu.SemaphoreType.DMA,
            pltpu.SemaphoreType.DMA,
            pltpu.SemaphoreType.DMA,
        ],
        compiler_params=pltpu.CompilerParams(use_tc_tiling_on_sc=False),
    )
    def gather_kernel(idx_hbm, table_hbm, out_hbm, idx_v, rows_v,
                      gsem0, gsem1, ssem0, ssem1):
        gsems = (gsem0, gsem1)
        ssems = (ssem0, ssem1)
        wid = lax.axis_index("s") * NC + lax.axis_index("c")
        base = pl.multiple_of(wid * b_per_w, 8)
        pltpu.sync_copy(idx_hbm.at[pl.ds(wid * n_chunks, n_chunks)], idx_v)

        def gather(g, b):
            return pltpu.make_async_copy(
                table_hbm.at[idx_v.at[g]], rows_v.at[b], gsems[b])

        def store(g, b):
            off = pl.multiple_of(base + g * C, 8)
            return pltpu.make_async_copy(
                rows_v.at[b], out_hbm.at[pl.ds(off, C)], ssems[b])

        for b in range(2):
            gather(b, b).start()

        @pl.loop(0, n_chunks - 2, step=2)
        def _chunks(g0):
            for b in range(2):
                g = g0 + b
                gather(g, b).wait()
                store(g, b).start()
                store(g, b).wait()
                gather(g + 2, b).start()

        for b in range(2):
            g = n_chunks - 2 + b
            gather(g, b).wait()
            store(g, b).start()
        for b in range(2):
            store(n_chunks - 2 + b, b).wait()

    return gather_kernel


def kernel(indices, table):
    B, H = indices.shape
    V, D = table.shape
    C = 1600
    flat_idx = indices.reshape(B * H // C, C).astype(jnp.int32)
    out = _make_gather(B * H, D, C)(flat_idx, table)
    return out.reshape(B, H, D)


# speedup vs baseline: 1.7798x; 1.6004x over previous
import functools

import jax
import jax.numpy as jnp
from jax import lax
from jax.experimental import pallas as pl
from jax.experimental.pallas import tpu as pltpu
from jax.experimental.pallas import tpu_sc as plsc


@functools.lru_cache(maxsize=None)
def _make_gather(B, H, D, R):
    info = plsc.get_sparse_core_info()
    NC = info.num_cores
    NW = NC * info.num_subcores
    assert B % (NW * R) == 0
    rows_per_w = B // NW
    n_chunks = rows_per_w // R
    assert n_chunks >= 2 and n_chunks % 2 == 0
    mesh = plsc.VectorSubcoreMesh(core_axis_name="c", subcore_axis_name="s")

    @functools.partial(
        pl.kernel,
        mesh=mesh,
        out_type=jax.ShapeDtypeStruct((B, H, D), jnp.float32),
        scratch_types=[
            pltpu.VMEM((2, R, H), jnp.int32),
            pltpu.VMEM((2, R, H, D), jnp.float32),
            pltpu.SemaphoreType.DMA,
            pltpu.SemaphoreType.DMA,
            pltpu.SemaphoreType.DMA,
            pltpu.SemaphoreType.DMA,
        ],
        compiler_params=pltpu.CompilerParams(use_tc_tiling_on_sc=False),
    )
    def gather_kernel(idx_hbm, table_hbm, out_hbm, idx_v, rows_v,
                      gsem0, gsem1, ssem0, ssem1):
        gsems = (gsem0, gsem1)
        ssems = (ssem0, ssem1)
        wid = lax.axis_index("s") * NC + lax.axis_index("c")
        base = wid * rows_per_w

        def stage_idx(g, b):
            pltpu.sync_copy(idx_hbm.at[pl.ds(base + g * R, R)], idx_v.at[b])

        def row_gather(b, i):
            return pltpu.make_async_copy(
                table_hbm.at[idx_v.at[b, i]], rows_v.at[b, i], gsems[b])

        def start_gathers(b):
            for i in range(R):
                row_gather(b, i).start()

        def wait_gathers(b):
            for i in range(R):
                row_gather(b, i).wait()

        def store(g, b):
            return pltpu.make_async_copy(
                rows_v.at[b], out_hbm.at[pl.ds(base + g * R, R)], ssems[b])

        for b in range(2):
            stage_idx(b, b)
            start_gathers(b)

        @pl.loop(0, n_chunks - 2, step=2)
        def _chunks(g0):
            for b in range(2):
                g = g0 + b
                wait_gathers(b)
                store(g, b).start()
                store(g, b).wait()
                stage_idx(g + 2, b)
                start_gathers(b)

        for b in range(2):
            g = n_chunks - 2 + b
            wait_gathers(b)
            store(g, b).start()
        for b in range(2):
            store(n_chunks - 2 + b, b).wait()

    return gather_kernel


def kernel(indices, table):
    B, H = indices.shape
    V, D = table.shape
    return _make_gather(B, H, D, 16)(indices.astype(jnp.int32), table)


# speedup vs baseline: 1.8006x; 1.0117x over previous
import functools

import jax
import jax.numpy as jnp
from jax import lax
from jax.experimental import pallas as pl
from jax.experimental.pallas import tpu as pltpu
from jax.experimental.pallas import tpu_sc as plsc


@functools.lru_cache(maxsize=None)
def _make_gather(B, H, D, R):
    info = plsc.get_sparse_core_info()
    NC = info.num_cores
    NW = NC * info.num_subcores
    assert B % (NW * R) == 0
    rows_per_w = B // NW
    n_chunks = rows_per_w // R
    assert n_chunks >= 2 and n_chunks % 2 == 0
    mesh = plsc.VectorSubcoreMesh(core_axis_name="c", subcore_axis_name="s")

    @functools.partial(
        pl.kernel,
        mesh=mesh,
        out_type=jax.ShapeDtypeStruct((B, H, D), jnp.float32),
        scratch_types=[
            pltpu.VMEM((2, R, H), jnp.int32),
            pltpu.VMEM((2, R, H, D), jnp.float32),
            pltpu.SemaphoreType.DMA,
            pltpu.SemaphoreType.DMA,
            pltpu.SemaphoreType.DMA,
            pltpu.SemaphoreType.DMA,
        ],
        compiler_params=pltpu.CompilerParams(use_tc_tiling_on_sc=False),
    )
    def gather_kernel(idx_hbm, table_hbm, out_hbm, idx_v, rows_v,
                      gsem0, gsem1, ssem0, ssem1):
        gsems = (gsem0, gsem1)
        ssems = (ssem0, ssem1)
        wid = lax.axis_index("s") * NC + lax.axis_index("c")
        base = wid * rows_per_w

        def stage_idx(g, b):
            pltpu.sync_copy(idx_hbm.at[pl.ds(base + g * R, R)], idx_v.at[b])

        def row_gather(b, i):
            return pltpu.make_async_copy(
                table_hbm.at[idx_v.at[b, i]], rows_v.at[b, i], gsems[b])

        def start_gathers(b):
            for i in range(R):
                row_gather(b, i).start()

        def wait_gathers(b):
            for i in range(R):
                row_gather(b, i).wait()

        def store(g, b):
            return pltpu.make_async_copy(
                rows_v.at[b], out_hbm.at[pl.ds(base + g * R, R)], ssems[b])

        for b in range(2):
            stage_idx(b, b)
            start_gathers(b)

        @pl.loop(0, n_chunks - 2, step=2)
        def _chunks(g0):
            for b in range(2):
                g = g0 + b
                wait_gathers(b)
                store(g, b).start()
                store(g, b).wait()
                stage_idx(g + 2, b)
                start_gathers(b)

        for b in range(2):
            g = n_chunks - 2 + b
            wait_gathers(b)
            store(g, b).start()
        for b in range(2):
            store(n_chunks - 2 + b, b).wait()

    return gather_kernel


def kernel(indices, table):
    B, H = indices.shape
    V, D = table.shape
    table_r = jnp.pad(table, ((0, 0), (0, 128 - D))).reshape(4 * V, D)
    idx4 = indices.astype(jnp.int32) * 4
    return _make_gather(B, H, D, 16)(idx4, table_r)
